# baseline (device time: 135500 ns/iter reference)
import jax
import jax.numpy as jnp
from jax import lax
from jax.experimental import pallas as pl
from jax.experimental.pallas import tpu as pltpu

N_DEV = 8
EPD = 4
N_EXP = N_DEV * EPD
CAP = 96
D = 1024
F = 2048
BLK = EPD * CAP
ROWS = N_DEV * CAP


def _body(s_ref, w1_ref, w2_ref, out_ref,
          r2, obig, w1f, w2f, w1v, w2v,
          local_sems, p1_send, p1_recv, p2_send, p2_recv):
    my = lax.axis_index("i")

    bsem = pltpu.get_barrier_semaphore()
    for o in range(1, N_DEV):
        pl.semaphore_signal(
            bsem, inc=1,
            device_id=(lax.rem(my + o, N_DEV),),
            device_id_type=pl.DeviceIdType.MESH,
        )
    pl.semaphore_wait(bsem, N_DEV - 1)

    p1_descs = []
    for j in range(EPD):
        for o in range(1, N_DEV):
            dst = lax.rem(my + o, N_DEV)
            d = pltpu.make_async_remote_copy(
                src_ref=s_ref.at[pl.ds((dst * EPD + j) * CAP, CAP)],
                dst_ref=r2.at[pl.ds(j * ROWS + my * CAP, CAP)],
                send_sem=p1_send.at[j, o],
                recv_sem=p1_recv.at[j, o],
                device_id=(dst,),
                device_id_type=pl.DeviceIdType.MESH,
            )
            d.start()
            p1_descs.append(d)

    for j in range(EPD):
        cp = pltpu.make_async_copy(
            s_ref.at[pl.ds((my * EPD + j) * CAP, CAP)],
            r2.at[pl.ds(j * ROWS + my * CAP, CAP)],
            local_sems.at[j],
        )
        cp.start()

    pltpu.make_async_copy(w1_ref.at[0], w1f.at[0], local_sems.at[4]).start()
    pltpu.make_async_copy(w2_ref.at[0], w2f.at[0], local_sems.at[5]).start()

    p2_descs = []
    for j in range(EPD):
        pltpu.make_async_copy(w1_ref.at[j], w1f.at[j % 2],
                              local_sems.at[4]).wait()
        pltpu.make_async_copy(w2_ref.at[j], w2f.at[j % 2],
                              local_sems.at[5]).wait()
        if j + 1 < EPD:
            pltpu.make_async_copy(w1_ref.at[j + 1], w1f.at[(j + 1) % 2],
                                  local_sems.at[4]).start()
            pltpu.make_async_copy(w2_ref.at[j + 1], w2f.at[(j + 1) % 2],
                                  local_sems.at[5]).start()
        w1v[j % 2] = w1f[j % 2].astype(jnp.bfloat16)
        w2v[j % 2] = w2f[j % 2].astype(jnp.bfloat16)

        pltpu.make_async_copy(
            s_ref.at[pl.ds((my * EPD + j) * CAP, CAP)],
            r2.at[pl.ds(j * ROWS + my * CAP, CAP)],
            local_sems.at[j],
        ).wait()
        for o in range(1, N_DEV):
            src = lax.rem(my - o + N_DEV, N_DEV)
            rd = pltpu.make_async_remote_copy(
                src_ref=s_ref.at[pl.ds(0, CAP)],
                dst_ref=r2.at[pl.ds(j * ROWS + src * CAP, CAP)],
                send_sem=p1_send.at[j, o],
                recv_sem=p1_recv.at[j, o],
                device_id=(src,),
                device_id_type=pl.DeviceIdType.MESH,
            )
            rd.wait_recv()

        rj = r2[pl.ds(j * ROWS, ROWS), :]
        h = jnp.dot(rj, w1v[j % 2], preferred_element_type=jnp.float32)
        h = jnp.maximum(h, 0.0).astype(jnp.bfloat16)
        oj = jnp.dot(h, w2v[j % 2], preferred_element_type=jnp.float32)
        obig[pl.ds(j * ROWS, ROWS), :] = oj.astype(jnp.bfloat16)

        for o in range(1, N_DEV):
            dst = lax.rem(my + o, N_DEV)
            d = pltpu.make_async_remote_copy(
                src_ref=obig.at[pl.ds(j * ROWS + dst * CAP, CAP)],
                dst_ref=out_ref.at[pl.ds((my * EPD + j) * CAP, CAP)],
                send_sem=p2_send.at[j, o],
                recv_sem=p2_recv.at[j, o],
                device_id=(dst,),
                device_id_type=pl.DeviceIdType.MESH,
            )
            d.start()
            p2_descs.append(d)
        cpo = pltpu.make_async_copy(
            obig.at[pl.ds(j * ROWS + my * CAP, CAP)],
            out_ref.at[pl.ds((my * EPD + j) * CAP, CAP)],
            local_sems.at[j],
        )
        cpo.start()
        cpo.wait()

    for j in range(EPD):
        for o in range(1, N_DEV):
            src = lax.rem(my - o + N_DEV, N_DEV)
            rd = pltpu.make_async_remote_copy(
                src_ref=obig.at[pl.ds(0, CAP)],
                dst_ref=out_ref.at[pl.ds((src * EPD + j) * CAP, CAP)],
                send_sem=p2_send.at[j, o],
                recv_sem=p2_recv.at[j, o],
                device_id=(src,),
                device_id_type=pl.DeviceIdType.MESH,
            )
            rd.wait_recv()

    for d in p1_descs:
        d.wait_send()
    for d in p2_descs:
        d.wait_send()


def kernel(x, assign, W1, W2):
    t, d = x.shape
    assign = assign.astype(jnp.int32)

    onehot = (assign[:, None]
              == jnp.arange(N_EXP, dtype=jnp.int32)[None, :]).astype(jnp.int32)
    rank = jnp.sum(onehot * (jnp.cumsum(onehot, axis=0) - 1), axis=1)
    slot = assign * CAP + rank
    slot = jnp.where(rank < CAP, slot, N_EXP * CAP)

    s_buf = jnp.zeros((N_EXP * CAP, d), jnp.bfloat16).at[slot].set(
        x.astype(jnp.bfloat16), mode="drop", unique_indices=True
    )

    ret = pl.pallas_call(
        _body,
        out_shape=jax.ShapeDtypeStruct((N_EXP * CAP, d), jnp.bfloat16),
        in_specs=[
            pl.BlockSpec(memory_space=pl.ANY),
            pl.BlockSpec(memory_space=pl.ANY),
            pl.BlockSpec(memory_space=pl.ANY),
        ],
        out_specs=pl.BlockSpec(memory_space=pl.ANY),
        scratch_shapes=[
            pltpu.VMEM((EPD * ROWS, D), jnp.bfloat16),
            pltpu.VMEM((EPD * ROWS, D), jnp.bfloat16),
            pltpu.VMEM((2, D, F), jnp.float32),
            pltpu.VMEM((2, F, D), jnp.float32),
            pltpu.VMEM((2, D, F), jnp.bfloat16),
            pltpu.VMEM((2, F, D), jnp.bfloat16),
            pltpu.SemaphoreType.DMA((6,)),
            pltpu.SemaphoreType.DMA((EPD, N_DEV)),
            pltpu.SemaphoreType.DMA((EPD, N_DEV)),
            pltpu.SemaphoreType.DMA((EPD, N_DEV)),
            pltpu.SemaphoreType.DMA((EPD, N_DEV)),
        ],
        compiler_params=pltpu.CompilerParams(
            collective_id=0, vmem_limit_bytes=100 * 1024 * 1024
        ),
    )(s_buf, W1, W2)

    inv = (t + jnp.arange(N_EXP * CAP, dtype=jnp.int32)).at[slot].set(
        jnp.arange(t, dtype=jnp.int32), mode="drop", unique_indices=True
    )
    out = jnp.zeros((t, d), jnp.bfloat16).at[inv].set(
        ret, mode="drop", unique_indices=True
    )
    return out.astype(jnp.float32)


# device time: 126083 ns/iter; 1.0747x vs baseline; 1.0747x over previous
import jax
import jax.numpy as jnp
from jax import lax
from jax.experimental import pallas as pl
from jax.experimental.pallas import tpu as pltpu

N_DEV = 8
EPD = 4
N_EXP = N_DEV * EPD
CAP = 96
D = 1024
F = 2048
BLK = EPD * CAP
ROWS = N_DEV * CAP


def _body(s_ref, w1_ref, w2_ref, out_ref,
          r2, obig, w1f, w2f, w1v, w2v,
          local_sems, p1_send, p1_recv, p2_send, p2_recv):
    my = lax.axis_index("i")

    bsem = pltpu.get_barrier_semaphore()
    for o in range(1, N_DEV):
        pl.semaphore_signal(
            bsem, inc=1,
            device_id=(lax.rem(my + o, N_DEV),),
            device_id_type=pl.DeviceIdType.MESH,
        )
    pl.semaphore_wait(bsem, N_DEV - 1)

    p1_descs = []
    for j in range(EPD):
        for o in range(1, N_DEV):
            dst = lax.rem(my + o, N_DEV)
            d = pltpu.make_async_remote_copy(
                src_ref=s_ref.at[pl.ds((dst * EPD + j) * CAP, CAP)],
                dst_ref=r2.at[pl.ds(j * ROWS + my * CAP, CAP)],
                send_sem=p1_send.at[j, o],
                recv_sem=p1_recv.at[j, o],
                device_id=(dst,),
                device_id_type=pl.DeviceIdType.MESH,
            )
            d.start()
            p1_descs.append(d)

    for j in range(EPD):
        cp = pltpu.make_async_copy(
            s_ref.at[pl.ds((my * EPD + j) * CAP, CAP)],
            r2.at[pl.ds(j * ROWS + my * CAP, CAP)],
            local_sems.at[j],
        )
        cp.start()

    pltpu.make_async_copy(w1_ref.at[0], w1f.at[0], local_sems.at[4]).start()
    pltpu.make_async_copy(w2_ref.at[0], w2f.at[0], local_sems.at[5]).start()

    p2_descs = []
    for j in range(EPD):
        pltpu.make_async_copy(w1_ref.at[j], w1f.at[j % 2],
                              local_sems.at[4]).wait()
        pltpu.make_async_copy(w2_ref.at[j], w2f.at[j % 2],
                              local_sems.at[5]).wait()
        if j + 1 < EPD:
            pltpu.make_async_copy(w1_ref.at[j + 1], w1f.at[(j + 1) % 2],
                                  local_sems.at[4]).start()
            pltpu.make_async_copy(w2_ref.at[j + 1], w2f.at[(j + 1) % 2],
                                  local_sems.at[5]).start()
        w1v[j % 2] = w1f[j % 2].astype(jnp.bfloat16)
        w2v[j % 2] = w2f[j % 2].astype(jnp.bfloat16)

        pltpu.make_async_copy(
            s_ref.at[pl.ds((my * EPD + j) * CAP, CAP)],
            r2.at[pl.ds(j * ROWS + my * CAP, CAP)],
            local_sems.at[j],
        ).wait()
        for o in range(1, N_DEV):
            src = lax.rem(my - o + N_DEV, N_DEV)
            rd = pltpu.make_async_remote_copy(
                src_ref=s_ref.at[pl.ds(0, CAP)],
                dst_ref=r2.at[pl.ds(j * ROWS + src * CAP, CAP)],
                send_sem=p1_send.at[j, o],
                recv_sem=p1_recv.at[j, o],
                device_id=(src,),
                device_id_type=pl.DeviceIdType.MESH,
            )
            rd.wait_recv()

        rj = r2[pl.ds(j * ROWS, ROWS), :]
        h = jnp.dot(rj, w1v[j % 2], preferred_element_type=jnp.float32)
        h = jnp.maximum(h, 0.0).astype(jnp.bfloat16)
        oj = jnp.dot(h, w2v[j % 2], preferred_element_type=jnp.float32)
        obig[pl.ds(j * ROWS, ROWS), :] = oj.astype(jnp.bfloat16)

        for o in range(1, N_DEV):
            dst = lax.rem(my + o, N_DEV)
            d = pltpu.make_async_remote_copy(
                src_ref=obig.at[pl.ds(j * ROWS + dst * CAP, CAP)],
                dst_ref=out_ref.at[pl.ds((my * EPD + j) * CAP, CAP)],
                send_sem=p2_send.at[j, o],
                recv_sem=p2_recv.at[j, o],
                device_id=(dst,),
                device_id_type=pl.DeviceIdType.MESH,
            )
            d.start()
            p2_descs.append(d)
        cpo = pltpu.make_async_copy(
            obig.at[pl.ds(j * ROWS + my * CAP, CAP)],
            out_ref.at[pl.ds((my * EPD + j) * CAP, CAP)],
            local_sems.at[j],
        )
        cpo.start()
        cpo.wait()

    for j in range(EPD):
        for o in range(1, N_DEV):
            src = lax.rem(my - o + N_DEV, N_DEV)
            rd = pltpu.make_async_remote_copy(
                src_ref=obig.at[pl.ds(0, CAP)],
                dst_ref=out_ref.at[pl.ds((src * EPD + j) * CAP, CAP)],
                send_sem=p2_send.at[j, o],
                recv_sem=p2_recv.at[j, o],
                device_id=(src,),
                device_id_type=pl.DeviceIdType.MESH,
            )
            rd.wait_recv()

    for d in p1_descs:
        d.wait_send()
    for d in p2_descs:
        d.wait_send()


def _prep_body(a_ref, slot_ref):
    a = a_ref[...]
    t = a.shape[0]
    iota = lax.broadcasted_iota(jnp.int32, (t, N_EXP), 1)
    oh = (a == iota).astype(jnp.int32)
    c = oh
    k = 1
    while k < t:
        c = c + jnp.concatenate(
            [jnp.zeros((k, N_EXP), jnp.int32), c[:-k]], axis=0
        )
        k *= 2
    rank = jnp.sum((c - 1) * oh, axis=1, keepdims=True)
    slot = a * CAP + rank
    slot_ref[...] = jnp.where(rank < CAP, slot, N_EXP * CAP)


def kernel(x, assign, W1, W2):
    t, d = x.shape
    assign = assign.astype(jnp.int32)

    slot = pl.pallas_call(
        _prep_body,
        out_shape=jax.ShapeDtypeStruct((t, 1), jnp.int32),
    )(assign.reshape(t, 1))[:, 0]

    s_buf = jnp.zeros((N_EXP * CAP, d), jnp.bfloat16).at[slot].set(
        x.astype(jnp.bfloat16), mode="drop", unique_indices=True
    )

    ret = pl.pallas_call(
        _body,
        out_shape=jax.ShapeDtypeStruct((N_EXP * CAP, d), jnp.bfloat16),
        in_specs=[
            pl.BlockSpec(memory_space=pl.ANY),
            pl.BlockSpec(memory_space=pl.ANY),
            pl.BlockSpec(memory_space=pl.ANY),
        ],
        out_specs=pl.BlockSpec(memory_space=pl.ANY),
        scratch_shapes=[
            pltpu.VMEM((EPD * ROWS, D), jnp.bfloat16),
            pltpu.VMEM((EPD * ROWS, D), jnp.bfloat16),
            pltpu.VMEM((2, D, F), jnp.float32),
            pltpu.VMEM((2, F, D), jnp.float32),
            pltpu.VMEM((2, D, F), jnp.bfloat16),
            pltpu.VMEM((2, F, D), jnp.bfloat16),
            pltpu.SemaphoreType.DMA((6,)),
            pltpu.SemaphoreType.DMA((EPD, N_DEV)),
            pltpu.SemaphoreType.DMA((EPD, N_DEV)),
            pltpu.SemaphoreType.DMA((EPD, N_DEV)),
            pltpu.SemaphoreType.DMA((EPD, N_DEV)),
        ],
        compiler_params=pltpu.CompilerParams(
            collective_id=0, vmem_limit_bytes=100 * 1024 * 1024
        ),
    )(s_buf, W1, W2)

    inv = (t + jnp.arange(N_EXP * CAP, dtype=jnp.int32)).at[slot].set(
        jnp.arange(t, dtype=jnp.int32), mode="drop", unique_indices=True
    )
    out = jnp.zeros((t, d), jnp.bfloat16).at[inv].set(
        ret, mode="drop", unique_indices=True
    )
    return out.astype(jnp.float32)


# device time: 118190 ns/iter; 1.1465x vs baseline; 1.0668x over previous
import jax
import jax.numpy as jnp
from jax import lax
from jax.experimental import pallas as pl
from jax.experimental.pallas import tpu as pltpu

N_DEV = 8
EPD = 4
N_EXP = N_DEV * EPD
CAP = 96
D = 1024
F = 2048
BLK = EPD * CAP
ROWS = N_DEV * CAP


def _body(s_ref, w1_ref, w2_ref, out_ref,
          r2, obig, w1f, w2f, w1v, w2v,
          local_sems, p1_send, p1_recv, p2_send, p2_recv):
    my = lax.axis_index("i")

    bsem = pltpu.get_barrier_semaphore()
    for o in range(1, N_DEV):
        pl.semaphore_signal(
            bsem, inc=1,
            device_id=(lax.rem(my + o, N_DEV),),
            device_id_type=pl.DeviceIdType.MESH,
        )
    pl.semaphore_wait(bsem, N_DEV - 1)

    p1_descs = []
    for j in range(EPD):
        for o in range(1, N_DEV):
            dst = lax.rem(my + o, N_DEV)
            d = pltpu.make_async_remote_copy(
                src_ref=s_ref.at[pl.ds((dst * EPD + j) * CAP, CAP)],
                dst_ref=r2.at[pl.ds(j * ROWS + my * CAP, CAP)],
                send_sem=p1_send.at[j, o],
                recv_sem=p1_recv.at[j, o],
                device_id=(dst,),
                device_id_type=pl.DeviceIdType.MESH,
            )
            d.start()
            p1_descs.append(d)

    for j in range(EPD):
        cp = pltpu.make_async_copy(
            s_ref.at[pl.ds((my * EPD + j) * CAP, CAP)],
            r2.at[pl.ds(j * ROWS + my * CAP, CAP)],
            local_sems.at[j],
        )
        cp.start()

    pltpu.make_async_copy(w1_ref.at[0], w1f.at[0], local_sems.at[4]).start()
    pltpu.make_async_copy(w2_ref.at[0], w2f.at[0], local_sems.at[5]).start()

    p2_descs = []
    for j in range(EPD):
        pltpu.make_async_copy(w1_ref.at[j], w1f.at[j % 2],
                              local_sems.at[4]).wait()
        pltpu.make_async_copy(w2_ref.at[j], w2f.at[j % 2],
                              local_sems.at[5]).wait()
        if j + 1 < EPD:
            pltpu.make_async_copy(w1_ref.at[j + 1], w1f.at[(j + 1) % 2],
                                  local_sems.at[4]).start()
            pltpu.make_async_copy(w2_ref.at[j + 1], w2f.at[(j + 1) % 2],
                                  local_sems.at[5]).start()
        w1v[j % 2] = w1f[j % 2].astype(jnp.bfloat16)
        w2v[j % 2] = w2f[j % 2].astype(jnp.bfloat16)

        pltpu.make_async_copy(
            s_ref.at[pl.ds((my * EPD + j) * CAP, CAP)],
            r2.at[pl.ds(j * ROWS + my * CAP, CAP)],
            local_sems.at[j],
        ).wait()
        for o in range(1, N_DEV):
            src = lax.rem(my - o + N_DEV, N_DEV)
            rd = pltpu.make_async_remote_copy(
                src_ref=s_ref.at[pl.ds(0, CAP)],
                dst_ref=r2.at[pl.ds(j * ROWS + src * CAP, CAP)],
                send_sem=p1_send.at[j, o],
                recv_sem=p1_recv.at[j, o],
                device_id=(src,),
                device_id_type=pl.DeviceIdType.MESH,
            )
            rd.wait_recv()

        rj = r2[pl.ds(j * ROWS, ROWS), :]
        h = jnp.dot(rj, w1v[j % 2], preferred_element_type=jnp.float32)
        h = jnp.maximum(h, 0.0).astype(jnp.bfloat16)
        oj = jnp.dot(h, w2v[j % 2], preferred_element_type=jnp.float32)
        obig[pl.ds(j * ROWS, ROWS), :] = oj.astype(jnp.bfloat16)

        for o in range(1, N_DEV):
            dst = lax.rem(my + o, N_DEV)
            d = pltpu.make_async_remote_copy(
                src_ref=obig.at[pl.ds(j * ROWS + dst * CAP, CAP)],
                dst_ref=out_ref.at[pl.ds((my * EPD + j) * CAP, CAP)],
                send_sem=p2_send.at[j, o],
                recv_sem=p2_recv.at[j, o],
                device_id=(dst,),
                device_id_type=pl.DeviceIdType.MESH,
            )
            d.start()
            p2_descs.append(d)
        cpo = pltpu.make_async_copy(
            obig.at[pl.ds(j * ROWS + my * CAP, CAP)],
            out_ref.at[pl.ds((my * EPD + j) * CAP, CAP)],
            local_sems.at[j],
        )
        cpo.start()
        cpo.wait()

    for j in range(EPD):
        for o in range(1, N_DEV):
            src = lax.rem(my - o + N_DEV, N_DEV)
            rd = pltpu.make_async_remote_copy(
                src_ref=obig.at[pl.ds(0, CAP)],
                dst_ref=out_ref.at[pl.ds((src * EPD + j) * CAP, CAP)],
                send_sem=p2_send.at[j, o],
                recv_sem=p2_recv.at[j, o],
                device_id=(src,),
                device_id_type=pl.DeviceIdType.MESH,
            )
            rd.wait_recv()

    for d in p1_descs:
        d.wait_send()
    for d in p2_descs:
        d.wait_send()


def _prep_body(a_ref, slot_ref, inv_ref):
    a = a_ref[...]
    t = a.shape[0]
    iota = lax.broadcasted_iota(jnp.int32, (t, N_EXP), 1)
    oh = (a == iota).astype(jnp.int32)
    c = oh
    k = 1
    while k < t:
        c = c + jnp.concatenate(
            [jnp.zeros((k, N_EXP), jnp.int32), c[:-k]], axis=0
        )
        k *= 2
    rank = jnp.sum((c - 1) * oh, axis=1, keepdims=True)
    slot = a * CAP + rank
    slot_ref[...] = jnp.where(rank < CAP, slot, N_EXP * CAP)

    ohf = oh.astype(jnp.float32)
    kiota = lax.broadcasted_iota(jnp.int32, (t, CAP), 1)
    ohk = (rank == kiota).astype(jnp.float32)
    tid = lax.broadcasted_iota(jnp.int32, (t, CAP), 0).astype(jnp.float32)
    dims = (((0,), (0,)), ((), ()))
    invf = lax.dot_general(ohf, ohk * tid, dims,
                           preferred_element_type=jnp.float32)
    cnt = lax.dot_general(ohf, ohk, dims,
                          preferred_element_type=jnp.float32)
    inv_ref[...] = jnp.where(cnt == 1.0, invf, float(t)).astype(jnp.int32)


def kernel(x, assign, W1, W2):
    t, d = x.shape
    assign = assign.astype(jnp.int32)

    slot2, inv2 = pl.pallas_call(
        _prep_body,
        out_shape=(
            jax.ShapeDtypeStruct((t, 1), jnp.int32),
            jax.ShapeDtypeStruct((N_EXP, CAP), jnp.int32),
        ),
    )(assign.reshape(t, 1))
    slot = slot2[:, 0]
    inv = inv2.reshape(N_EXP * CAP)

    s_buf = jnp.zeros((N_EXP * CAP, d), jnp.bfloat16).at[slot].set(
        x.astype(jnp.bfloat16), mode="drop"
    )

    ret = pl.pallas_call(
        _body,
        out_shape=jax.ShapeDtypeStruct((N_EXP * CAP, d), jnp.bfloat16),
        in_specs=[
            pl.BlockSpec(memory_space=pl.ANY),
            pl.BlockSpec(memory_space=pl.ANY),
            pl.BlockSpec(memory_space=pl.ANY),
        ],
        out_specs=pl.BlockSpec(memory_space=pl.ANY),
        scratch_shapes=[
            pltpu.VMEM((EPD * ROWS, D), jnp.bfloat16),
            pltpu.VMEM((EPD * ROWS, D), jnp.bfloat16),
            pltpu.VMEM((2, D, F), jnp.float32),
            pltpu.VMEM((2, F, D), jnp.float32),
            pltpu.VMEM((2, D, F), jnp.bfloat16),
            pltpu.VMEM((2, F, D), jnp.bfloat16),
            pltpu.SemaphoreType.DMA((6,)),
            pltpu.SemaphoreType.DMA((EPD, N_DEV)),
            pltpu.SemaphoreType.DMA((EPD, N_DEV)),
            pltpu.SemaphoreType.DMA((EPD, N_DEV)),
            pltpu.SemaphoreType.DMA((EPD, N_DEV)),
        ],
        compiler_params=pltpu.CompilerParams(
            collective_id=0, vmem_limit_bytes=100 * 1024 * 1024
        ),
    )(s_buf, W1, W2)

    out = jnp.zeros((t, d), jnp.bfloat16).at[inv].set(ret, mode="drop")
    return out.astype(jnp.float32)
